# baseline (speedup 1.0000x reference)
import functools

import jax
import jax.numpy as jnp
from jax import lax
from jax.experimental import pallas as pl
from jax.experimental.pallas import tpu as pltpu
from jax.experimental.pallas import tpu_sc as plsc

_B = 16384
_EMB = 32
_H1 = 64
_NC = 2
_NS = 16
_NW = _NC * _NS
_BPW = _B // _NW
_CHUNK = 128
_NCHUNK = _BPW // _CHUNK

_MLP_BLK = 2048


def _gather_body(uids, iids, utab, itab, uout, iout, uidx, iidx, urows,
                 irows, sem):
    wid = lax.axis_index("s") * _NC + lax.axis_index("c")
    base = wid * _BPW
    pltpu.sync_copy(uids.at[wid], uidx)
    pltpu.sync_copy(iids.at[wid], iidx)
    copies = []
    for j in range(_NCHUNK):
        copies.append(
            pltpu.async_copy(utab.at[uidx.at[j]],
                             urows.at[pl.ds(j * _CHUNK, _CHUNK)], sem))
        copies.append(
            pltpu.async_copy(itab.at[iidx.at[j]],
                             irows.at[pl.ds(j * _CHUNK, _CHUNK)], sem))
    for cp in copies:
        cp.wait()
    pltpu.sync_copy(urows, uout.at[pl.ds(base, _BPW)])
    pltpu.sync_copy(irows, iout.at[pl.ds(base, _BPW)])


@jax.jit
def _gather(uids, iids, utab, itab):
    mesh = plsc.VectorSubcoreMesh(core_axis_name="c", subcore_axis_name="s")
    fn = functools.partial(
        pl.kernel,
        mesh=mesh,
        out_type=(
            jax.ShapeDtypeStruct((_B, _EMB), jnp.float32),
            jax.ShapeDtypeStruct((_B, _EMB), jnp.float32),
        ),
        scratch_types=[
            pltpu.VMEM((_NCHUNK, _CHUNK), jnp.int32),
            pltpu.VMEM((_NCHUNK, _CHUNK), jnp.int32),
            pltpu.VMEM((_BPW, _EMB), jnp.float32),
            pltpu.VMEM((_BPW, _EMB), jnp.float32),
            pltpu.SemaphoreType.DMA,
        ],
        compiler_params=pltpu.CompilerParams(use_tc_tiling_on_sc=False),
    )(_gather_body)
    return fn(uids, iids, utab, itab)


def _mlp_body(u_ref, i_ref, w1a_ref, w1b_ref, b1_ref, w2_ref, b2_ref,
              w3_ref, b3_ref, o_ref):
    u = u_ref[...]
    v = i_ref[...]
    h = jnp.dot(u, w1a_ref[...], preferred_element_type=jnp.float32)
    h = h + jnp.dot(v, w1b_ref[...], preferred_element_type=jnp.float32)
    h = jnp.maximum(h + b1_ref[...], 0.0)
    h = jnp.dot(h, w2_ref[...], preferred_element_type=jnp.float32)
    h = jnp.maximum(h + b2_ref[...], 0.0)
    logit = jnp.sum(h * w3_ref[...], axis=1) + b3_ref[0]
    o_ref[...] = 1.0 / (1.0 + jnp.exp(-logit))


@jax.jit
def _mlp(uemb, iemb, w1a, w1b, b1, w2, b2, w3, b3):
    grid = (_B // _MLP_BLK,)
    return pl.pallas_call(
        _mlp_body,
        grid=grid,
        in_specs=[
            pl.BlockSpec((_MLP_BLK, _EMB), lambda i: (i, 0)),
            pl.BlockSpec((_MLP_BLK, _EMB), lambda i: (i, 0)),
            pl.BlockSpec((_EMB, _H1), lambda i: (0, 0)),
            pl.BlockSpec((_EMB, _H1), lambda i: (0, 0)),
            pl.BlockSpec((1, _H1), lambda i: (0, 0)),
            pl.BlockSpec((_H1, _EMB), lambda i: (0, 0)),
            pl.BlockSpec((1, _EMB), lambda i: (0, 0)),
            pl.BlockSpec((1, _EMB), lambda i: (0, 0)),
            pl.BlockSpec(memory_space=pltpu.SMEM),
        ],
        out_specs=pl.BlockSpec((_MLP_BLK,), lambda i: (i,)),
        out_shape=jax.ShapeDtypeStruct((_B,), jnp.float32),
    )(uemb, iemb, w1a, w1b, b1, w2, b2, w3, b3)


def kernel(user_ids, item_ids, user_table, item_table, W1, b1, W2, b2, W3,
           b3):
    uids = user_ids.astype(jnp.int32).reshape(_NW, _NCHUNK, _CHUNK)
    iids = item_ids.astype(jnp.int32).reshape(_NW, _NCHUNK, _CHUNK)
    uemb, iemb = _gather(uids, iids, user_table, item_table)
    return _mlp(
        uemb, iemb,
        W1[:_EMB], W1[_EMB:],
        b1.reshape(1, _H1),
        W2,
        b2.reshape(1, _EMB),
        W3.reshape(1, _EMB),
        b3.reshape(1),
    )


# speedup vs baseline: 1.4718x; 1.4718x over previous
import functools

import jax
import jax.numpy as jnp
from jax import lax
from jax.experimental import pallas as pl
from jax.experimental.pallas import tpu as pltpu
from jax.experimental.pallas import tpu_sc as plsc

_B = 16384
_EMB = 32
_H1 = 64
_NROWS = 1000000
_SUB = 8
_NBLK = _NROWS // _SUB
_NC = 2
_NS = 16
_NW = _NC * _NS
_BPW = _B // _NW
_CH = 64
_NCHUNK = _BPW // _CH
_L = 16

_MLP_BLK = 2048


def _gather_body(uids, iids, utab, itab, uout, iout, uids_v, iids_v,
                 urows, irows, sem):
    wid = lax.axis_index("s") * _NC + lax.axis_index("c")
    base = wid * _BPW
    pltpu.sync_copy(uids.at[pl.ds(base, _BPW)], uids_v)
    pltpu.sync_copy(iids.at[pl.ds(base, _BPW)], iids_v)

    def chunk(c, _):
        copies = []
        for g in range(_CH // _L):
            u16 = uids_v[pl.ds(c * _CH + g * _L, _L)]
            i16 = iids_v[pl.ds(c * _CH + g * _L, _L)]
            for t in range(_L):
                su = u16[t]
                si = i16[t]
                row = g * _L + t
                copies.append(
                    pltpu.async_copy(utab.at[pl.ds(su, 1)],
                                     urows.at[pl.ds(row, 1)], sem))
                copies.append(
                    pltpu.async_copy(itab.at[pl.ds(si, 1)],
                                     irows.at[pl.ds(row, 1)], sem))
        for cp in copies:
            cp.wait()
        pltpu.sync_copy(urows, uout.at[pl.ds(base + c * _CH, _CH)])
        pltpu.sync_copy(irows, iout.at[pl.ds(base + c * _CH, _CH)])
        return 0

    lax.fori_loop(0, _NCHUNK, chunk, 0)


@jax.jit
def _gather(uids, iids, utab, itab):
    mesh = plsc.VectorSubcoreMesh(core_axis_name="c", subcore_axis_name="s")
    fn = functools.partial(
        pl.kernel,
        mesh=mesh,
        out_type=(
            jax.ShapeDtypeStruct((_B, _EMB), jnp.float32),
            jax.ShapeDtypeStruct((_B, _EMB), jnp.float32),
        ),
        scratch_types=[
            pltpu.VMEM((_BPW,), jnp.int32),
            pltpu.VMEM((_BPW,), jnp.int32),
            pltpu.VMEM((_CH, _EMB), jnp.float32),
            pltpu.VMEM((_CH, _EMB), jnp.float32),
            pltpu.SemaphoreType.DMA,
        ],
        compiler_params=pltpu.CompilerParams(needs_layout_passes=False),
    )(_gather_body)
    return fn(uids, iids, utab, itab)


def _mlp_body(u_ref, i_ref, w1a_ref, w1b_ref, b1_ref, w2_ref, b2_ref,
              w3_ref, b3_ref, o_ref):
    u = u_ref[...]
    v = i_ref[...]
    h = jnp.dot(u, w1a_ref[...], preferred_element_type=jnp.float32)
    h = h + jnp.dot(v, w1b_ref[...], preferred_element_type=jnp.float32)
    h = jnp.maximum(h + b1_ref[...], 0.0)
    h = jnp.dot(h, w2_ref[...], preferred_element_type=jnp.float32)
    h = jnp.maximum(h + b2_ref[...], 0.0)
    logit = jnp.sum(h * w3_ref[...], axis=1) + b3_ref[0]
    o_ref[...] = 1.0 / (1.0 + jnp.exp(-logit))


@jax.jit
def _mlp(uemb, iemb, w1a, w1b, b1, w2, b2, w3, b3):
    grid = (_B // _MLP_BLK,)
    return pl.pallas_call(
        _mlp_body,
        grid=grid,
        in_specs=[
            pl.BlockSpec((_MLP_BLK, _EMB), lambda i: (i, 0)),
            pl.BlockSpec((_MLP_BLK, _EMB), lambda i: (i, 0)),
            pl.BlockSpec((_EMB, _H1), lambda i: (0, 0)),
            pl.BlockSpec((_EMB, _H1), lambda i: (0, 0)),
            pl.BlockSpec((1, _H1), lambda i: (0, 0)),
            pl.BlockSpec((_H1, _EMB), lambda i: (0, 0)),
            pl.BlockSpec((1, _EMB), lambda i: (0, 0)),
            pl.BlockSpec((1, _EMB), lambda i: (0, 0)),
            pl.BlockSpec(memory_space=pltpu.SMEM),
        ],
        out_specs=pl.BlockSpec((_MLP_BLK,), lambda i: (i,)),
        out_shape=jax.ShapeDtypeStruct((_B,), jnp.float32),
    )(uemb, iemb, w1a, w1b, b1, w2, b2, w3, b3)


def kernel(user_ids, item_ids, user_table, item_table, W1, b1, W2, b2, W3,
           b3):
    uids = user_ids.astype(jnp.int32)
    iids = item_ids.astype(jnp.int32)
    uemb, iemb = _gather(uids, iids, user_table, item_table)
    return _mlp(
        uemb, iemb,
        W1[:_EMB], W1[_EMB:],
        b1.reshape(1, _H1),
        W2,
        b2.reshape(1, _EMB),
        W3.reshape(1, _EMB),
        b3.reshape(1),
    )


# speedup vs baseline: 3.3633x; 2.2852x over previous
import functools

import jax
import jax.numpy as jnp
from jax import lax
from jax.experimental import pallas as pl
from jax.experimental.pallas import tpu as pltpu
from jax.experimental.pallas import tpu_sc as plsc

_B = 16384
_EMB = 32
_H1 = 64
_NROWS = 1000000
_NC = 2
_NS = 16
_NW = _NC * _NS
_BPW = _B // _NW
_L = 16
_RING = 16
_LANES = 128

_MLP_BLK = 2048


def _gather_body(uids, iids, utab, itab, uout, iout, uids_v, iids_v,
                 ring, ubuf, ibuf, sem):
    wid = lax.axis_index("s") * _NC + lax.axis_index("c")
    base = pl.multiple_of(wid * _BPW, _BPW)
    pltpu.sync_copy(uids.at[pl.ds(base, _BPW)], uids_v)
    pltpu.sync_copy(iids.at[pl.ds(base, _BPW)], iids_v)

    lanes = lax.iota(jnp.int32, _L)

    def table_pass(tab, ids_v, obuf):
        def group(g, _):
            ids16 = ids_v[pl.ds(g * _L, _L)]
            rt16 = lax.shift_right_logical(ids16, 7)
            su16 = lax.bitwise_and(ids16, _LANES - 1)
            copies = []
            for t in range(_L):
                rt = rt16[t]
                off = pl.multiple_of(rt * _LANES, _LANES)
                copies.append(
                    pltpu.async_copy(tab.at[:, pl.ds(off, _LANES)],
                                     ring.at[t], sem))
            for cp in copies:
                cp.wait()
            for t in range(_L):
                su = su16[t]
                col = jnp.full((_L,), g * _L + t, jnp.int32)
                suv = jnp.full((_L,), su, jnp.int32)
                tv = jnp.full((_L,), t, jnp.int32)
                lo = plsc.load_gather(ring, [tv, lanes, suv])
                hi = plsc.load_gather(ring, [tv, lanes + _L, suv])
                plsc.store_scatter(obuf, [lanes, col], lo)
                plsc.store_scatter(obuf, [lanes + _L, col], hi)
            return 0

        lax.fori_loop(0, _BPW // _L, group, 0)

    table_pass(utab, uids_v, ubuf)
    table_pass(itab, iids_v, ibuf)
    pltpu.sync_copy(ubuf, uout.at[:, pl.ds(base, _BPW)])
    pltpu.sync_copy(ibuf, iout.at[:, pl.ds(base, _BPW)])


@jax.jit
def _gather(uids, iids, utab, itab):
    mesh = plsc.VectorSubcoreMesh(core_axis_name="c", subcore_axis_name="s")
    fn = functools.partial(
        pl.kernel,
        mesh=mesh,
        out_type=(
            jax.ShapeDtypeStruct((_EMB, _B), jnp.float32),
            jax.ShapeDtypeStruct((_EMB, _B), jnp.float32),
        ),
        scratch_types=[
            pltpu.VMEM((_BPW,), jnp.int32),
            pltpu.VMEM((_BPW,), jnp.int32),
            pltpu.VMEM((_RING, _EMB, _LANES), jnp.float32),
            pltpu.VMEM((_EMB, _BPW), jnp.float32),
            pltpu.VMEM((_EMB, _BPW), jnp.float32),
            pltpu.SemaphoreType.DMA,
        ],
        compiler_params=pltpu.CompilerParams(needs_layout_passes=False),
    )(_gather_body)
    return fn(uids, iids, utab, itab)


def _mlp_body(u_ref, i_ref, w1a_ref, w1b_ref, b1_ref, w2_ref, b2_ref,
              w3_ref, b3_ref, o_ref):
    u = u_ref[...]
    v = i_ref[...]
    h = jnp.dot(w1a_ref[...], u, preferred_element_type=jnp.float32)
    h = h + jnp.dot(w1b_ref[...], v, preferred_element_type=jnp.float32)
    h = jnp.maximum(h + b1_ref[...], 0.0)
    h = jnp.dot(w2_ref[...], h, preferred_element_type=jnp.float32)
    h = jnp.maximum(h + b2_ref[...], 0.0)
    logit = jnp.sum(h * w3_ref[...], axis=0) + b3_ref[0]
    o_ref[...] = 1.0 / (1.0 + jnp.exp(-logit))


@jax.jit
def _mlp(uembT, iembT, w1aT, w1bT, b1c, w2T, b2c, w3c, b3):
    grid = (_B // _MLP_BLK,)
    return pl.pallas_call(
        _mlp_body,
        grid=grid,
        in_specs=[
            pl.BlockSpec((_EMB, _MLP_BLK), lambda i: (0, i)),
            pl.BlockSpec((_EMB, _MLP_BLK), lambda i: (0, i)),
            pl.BlockSpec((_H1, _EMB), lambda i: (0, 0)),
            pl.BlockSpec((_H1, _EMB), lambda i: (0, 0)),
            pl.BlockSpec((_H1, 1), lambda i: (0, 0)),
            pl.BlockSpec((_EMB, _H1), lambda i: (0, 0)),
            pl.BlockSpec((_EMB, 1), lambda i: (0, 0)),
            pl.BlockSpec((_EMB, 1), lambda i: (0, 0)),
            pl.BlockSpec(memory_space=pltpu.SMEM),
        ],
        out_specs=pl.BlockSpec((_MLP_BLK,), lambda i: (i,)),
        out_shape=jax.ShapeDtypeStruct((_B,), jnp.float32),
    )(uembT, iembT, w1aT, w1bT, b1c, w2T, b2c, w3c, b3)


def kernel(user_ids, item_ids, user_table, item_table, W1, b1, W2, b2, W3,
           b3):
    uids = user_ids.astype(jnp.int32)
    iids = item_ids.astype(jnp.int32)
    uembT, iembT = _gather(uids, iids, user_table.T, item_table.T)
    return _mlp(
        uembT, iembT,
        W1[:_EMB].T, W1[_EMB:].T,
        b1.reshape(_H1, 1),
        W2.T,
        b2.reshape(_EMB, 1),
        W3.reshape(_EMB, 1),
        b3.reshape(1),
    )
